# baseline (speedup 1.0000x reference)
import functools

import jax
import jax.numpy as jnp
from jax import lax
from jax.experimental import pallas as pl
from jax.experimental.pallas import tpu as pltpu
from jax.experimental.pallas import tpu_sc as plsc

N = 10000
E = 160000
IN_DIM = 128
HID = 64
MSG = 16
EDGE_DIM = 16
EPS = 1e-5

NC = 2
NS = 16
NW = NC * NS
CHUNK = 125
CH_W = 40
SEG = 1
CH_S = CH_W // SEG
E_S = E // SEG
E_WS = CH_S * CHUNK
NPAD = 10016
ROWS_W = NPAD // NS

BE = 1600
BN = 1000

@functools.cache
def _mesh():
    return plsc.VectorSubcoreMesh(core_axis_name="c", subcore_axis_name="s",
                                  num_cores=NC, num_subcores=NS)


def _sc_compiler_params():
    return pltpu.CompilerParams(use_tc_tiling_on_sc=False)


def _sc_gather(h, src3):
    KB = 10

    @functools.partial(
        pl.kernel,
        out_type=jax.ShapeDtypeStruct((E_S, HID), jnp.bfloat16),
        mesh=_mesh(),
        compiler_params=_sc_compiler_params(),
        scratch_types=[
            pltpu.VMEM((CH_S, CHUNK), jnp.int32),
            pltpu.VMEM((KB, CHUNK, HID), jnp.bfloat16),
            pltpu.SemaphoreType.DMA,
            pltpu.SemaphoreType.DMA,
        ],
    )
    def k(h_hbm, src_hbm, out_hbm, idx_v, rows_v, sem, osem):
        wid = lax.axis_index("s") * NC + lax.axis_index("c")
        pltpu.sync_copy(src_hbm.at[wid], idx_v)

        def group(g, carry):
            j0 = g * KB
            descs = [
                pltpu.async_copy(h_hbm.at[idx_v.at[j0 + b]], rows_v.at[b],
                                 sem)
                for b in range(KB)
            ]
            outs = []
            for b in range(KB):
                descs[b].wait()
                outs.append(pltpu.async_copy(
                    rows_v.at[b],
                    out_hbm.at[pl.ds((wid * CH_S + j0 + b) * CHUNK, CHUNK)],
                    osem))
            for o in outs:
                o.wait()
            return carry

        lax.fori_loop(0, CH_S // KB, group, 0)

    return k(h, src3)


def _sc_scatter(m3, dst3, zz, ones, with_count):
    KB = 10

    out_type = [jax.ShapeDtypeStruct((NC, NPAD, MSG), jnp.float32)]
    scratch = [
        pltpu.VMEM((CH_S, CHUNK), jnp.int32),
        pltpu.VMEM((E_WS, MSG), jnp.float32),
        pltpu.VMEM((CHUNK, MSG), jnp.float32),
        pltpu.VMEM_SHARED((NPAD, MSG), jnp.float32),
    ]
    if with_count:
        out_type.append(jax.ShapeDtypeStruct((NC, NPAD, MSG), jnp.float32))
        scratch.append(pltpu.VMEM_SHARED((NPAD, MSG), jnp.float32))
    scratch.append(pltpu.SemaphoreType.DMA)

    @functools.partial(
        pl.kernel,
        out_type=tuple(out_type),
        mesh=_mesh(),
        compiler_params=_sc_compiler_params(),
        scratch_types=scratch,
    )
    def k(m_hbm, dst_hbm, zz_hbm, ones_hbm, *rest):
        if with_count:
            s_out, c_out, idx_v, m_v, ones_v, shared_s, shared_c, sem = rest
        else:
            (s_out, idx_v, m_v, ones_v, shared_s, sem) = rest
        cid = lax.axis_index("c")
        sid = lax.axis_index("s")
        wid = sid * NC + cid
        r0 = sid * ROWS_W
        pltpu.sync_copy(zz_hbm.at[pl.ds(r0, ROWS_W)],
                        shared_s.at[pl.ds(r0, ROWS_W)])
        if with_count:
            pltpu.sync_copy(zz_hbm.at[pl.ds(r0, ROWS_W)],
                            shared_c.at[pl.ds(r0, ROWS_W)])
        pltpu.sync_copy(ones_hbm, ones_v)
        plsc.subcore_barrier()
        pltpu.sync_copy(dst_hbm.at[wid], idx_v)
        pltpu.sync_copy(m_hbm.at[wid], m_v)

        def body(g, carry):
            j0 = g * KB
            descs = []
            for b in range(KB):
                descs.append(pltpu.async_copy(
                    m_v.at[pl.ds((j0 + b) * CHUNK, CHUNK)],
                    shared_s.at[idx_v.at[j0 + b]], sem, add=True))
                if with_count:
                    descs.append(pltpu.async_copy(
                        ones_v, shared_c.at[idx_v.at[j0 + b]], sem,
                        add=True))
            for d in descs:
                d.wait()
            return carry

        lax.fori_loop(0, CH_S // KB, body, 0)
        plsc.subcore_barrier()
        pltpu.sync_copy(shared_s.at[pl.ds(r0, ROWS_W)],
                        s_out.at[cid, pl.ds(r0, ROWS_W)])
        if with_count:
            pltpu.sync_copy(shared_c.at[pl.ds(r0, ROWS_W)],
                            c_out.at[cid, pl.ds(r0, ROWS_W)])

    return k(m3, dst3, zz, ones)


def _tc_input_proj(x, W, b):
    def body(x_ref, w_ref, b_ref, o_ref, ob_ref):
        h = x_ref[...] @ w_ref[...] + b_ref[...]
        o_ref[...] = h
        ob_ref[...] = h.astype(jnp.bfloat16)

    return pl.pallas_call(
        body,
        grid=(N // BN,),
        in_specs=[
            pl.BlockSpec((BN, IN_DIM), lambda i: (i, 0)),
            pl.BlockSpec((IN_DIM, HID), lambda i: (0, 0)),
            pl.BlockSpec((1, HID), lambda i: (0, 0)),
        ],
        out_specs=[pl.BlockSpec((BN, HID), lambda i: (i, 0)),
                   pl.BlockSpec((BN, HID), lambda i: (i, 0))],
        out_shape=[jax.ShapeDtypeStruct((N, HID), jnp.float32),
                   jax.ShapeDtypeStruct((N, HID), jnp.bfloat16)],
    )(x, W, b.reshape(1, HID))


def _tc_messages(ea, xj, e1W, e1b, e2Wp, b2r, sel, seg):
    def body(ea_ref, xj_ref, w1_ref, b1_ref, w2_ref, b2_ref, sel_ref, o_ref):
        u = jnp.maximum(ea_ref[...] @ w1_ref[...] + b1_ref[...], 0.0)
        wp = jnp.dot(u.astype(jnp.bfloat16), w2_ref[...],
                     preferred_element_type=jnp.float32
                     ).astype(jnp.bfloat16)
        xj = xj_ref[...]
        xt = jnp.tile(xj, (1, MSG))
        p = wp * xt
        o_ref[...] = (jnp.dot(p, sel_ref[...],
                              preferred_element_type=jnp.float32)
                      + jnp.dot(xj, b2_ref[...],
                                preferred_element_type=jnp.float32))

    nblk = E_S // BE
    return pl.pallas_call(
        body,
        grid=(nblk,),
        in_specs=[
            pl.BlockSpec((BE, EDGE_DIM), lambda i: (seg * nblk + i, 0)),
            pl.BlockSpec((BE, HID), lambda i: (i, 0)),
            pl.BlockSpec((EDGE_DIM, HID), lambda i: (0, 0)),
            pl.BlockSpec((1, HID), lambda i: (0, 0)),
            pl.BlockSpec((HID, MSG * HID), lambda i: (0, 0)),
            pl.BlockSpec((HID, MSG), lambda i: (0, 0)),
            pl.BlockSpec((MSG * HID, MSG), lambda i: (0, 0)),
        ],
        out_specs=pl.BlockSpec((BE, MSG), lambda i: (i, 0)),
        out_shape=jax.ShapeDtypeStruct((E_S, MSG), jnp.float32),
    )(ea, xj, e1W, e1b.reshape(1, HID), e2Wp.astype(jnp.bfloat16),
      b2r.astype(jnp.bfloat16), sel)


def _tc_node_update(h, s_part, c_part, rW, cb, mp_W, mp_b, scale, shift,
                    final_W=None, final_b=None):
    last = final_W is not None
    out_dim = final_W.shape[1] if last else HID
    np_ = s_part.shape[0]
    nc_ = c_part.shape[0]

    def body(h_ref, s_ref, c_ref, rw_ref, cb_ref, mw_ref, mb_ref,
             sc_ref, sh_ref, *rest):
        h = h_ref[...]
        s = jnp.sum(s_ref[...], axis=0)
        cnt = jnp.maximum(jnp.sum(c_ref[...], axis=0), 1.0)
        agg = s / cnt
        xm = agg + h @ rw_ref[...] + cb_ref[...]
        h2 = h + xm @ mw_ref[...] + mb_ref[...]
        h2 = jnp.maximum(h2 * sc_ref[...] + sh_ref[...], 0.0)
        if last:
            fw_ref, fb_ref, o_ref = rest
            o_ref[...] = h2 @ fw_ref[...] + fb_ref[...]
        else:
            o_ref, ob_ref = rest
            o_ref[...] = h2
            ob_ref[...] = h2.astype(jnp.bfloat16)

    in_specs = [
        pl.BlockSpec((BN, HID), lambda i: (i, 0)),
        pl.BlockSpec((np_, BN, MSG), lambda i: (0, i, 0)),
        pl.BlockSpec((nc_, BN, MSG), lambda i: (0, i, 0)),
        pl.BlockSpec((HID, MSG), lambda i: (0, 0)),
        pl.BlockSpec((1, MSG), lambda i: (0, 0)),
        pl.BlockSpec((MSG, HID), lambda i: (0, 0)),
        pl.BlockSpec((1, HID), lambda i: (0, 0)),
        pl.BlockSpec((1, HID), lambda i: (0, 0)),
        pl.BlockSpec((1, HID), lambda i: (0, 0)),
    ]
    args = [h, s_part, c_part, rW, cb.reshape(1, MSG), mp_W,
            mp_b.reshape(1, HID), scale.reshape(1, HID), shift.reshape(1, HID)]
    if last:
        in_specs.append(pl.BlockSpec((HID, out_dim), lambda i: (0, 0)))
        in_specs.append(pl.BlockSpec((1, out_dim), lambda i: (0, 0)))
        args.append(final_W)
        args.append(final_b.reshape(1, out_dim))

    if last:
        out_specs = pl.BlockSpec((BN, out_dim), lambda i: (i, 0))
        out_shape = jax.ShapeDtypeStruct((N, out_dim), jnp.float32)
    else:
        out_specs = [pl.BlockSpec((BN, HID), lambda i: (i, 0)),
                     pl.BlockSpec((BN, HID), lambda i: (i, 0))]
        out_shape = [jax.ShapeDtypeStruct((N, HID), jnp.float32),
                     jax.ShapeDtypeStruct((N, HID), jnp.bfloat16)]
    return pl.pallas_call(
        body,
        grid=(N // BN,),
        in_specs=in_specs,
        out_specs=out_specs,
        out_shape=out_shape,
    )(*args)


def _permute_e2(e2W, e2b):
    Wp = e2W.reshape(HID, HID, MSG).transpose(0, 2, 1).reshape(HID, MSG * HID)
    return Wp, e2b.reshape(HID, MSG)


def kernel(x, edge_index, edge_attr, ip_W, ip_b,
           e1W_0, e1b_0, e2W_0, e2b_0, rW_0, cb_0, g_0, be_0, rm_0, rv_0,
           e1W_1, e1b_1, e2W_1, e2b_1, rW_1, cb_1, g_1, be_1, rm_1, rv_1,
           mp_W, mp_b, op_W, op_b):
    src, dst = edge_index[0], edge_index[1]
    src4 = src.reshape(SEG, NW, CH_S, CHUNK)
    dst4 = dst.reshape(SEG, NW, CH_S, CHUNK)
    ea = edge_attr
    zz = jnp.zeros((NPAD, MSG), jnp.float32)
    ones = jnp.ones((CHUNK, MSG), jnp.float32)
    sel = jnp.repeat(jnp.eye(MSG, dtype=jnp.bfloat16), HID, axis=0)

    h, hb = _tc_input_proj(x, ip_W, ip_b)

    layers = [
        (e1W_0, e1b_0, e2W_0, e2b_0, rW_0, cb_0, g_0, be_0, rm_0, rv_0),
        (e1W_1, e1b_1, e2W_1, e2b_1, rW_1, cb_1, g_1, be_1, rm_1, rv_1),
    ]
    c_part = None
    for li, (e1W, e1b, e2W, e2b, rW, cb, g, be, rm, rv) in enumerate(layers):
        e2Wp, b2r = _permute_e2(e2W, e2b)
        scale = g / jnp.sqrt(rv + EPS)
        shift = be - rm * scale

        xj = _sc_gather(hb, src4[0])
        m = _tc_messages(ea, xj, e1W, e1b, e2Wp, b2r, sel, 0)
        m3 = m.reshape(NW, E_WS, MSG)
        if li == 0:
            s_part, c_part = _sc_scatter(m3, dst4[0], zz, ones,
                                         with_count=True)
        else:
            (s_part,) = _sc_scatter(m3, dst4[0], zz, ones,
                                    with_count=False)

        if li == 0:
            h, hb = _tc_node_update(h, s_part, c_part, rW, cb,
                                    mp_W, mp_b, scale, shift)
        else:
            h = _tc_node_update(h, s_part, c_part, rW, cb,
                                mp_W, mp_b, scale, shift,
                                final_W=op_W, final_b=op_b)
    return h


# speedup vs baseline: 1.0853x; 1.0853x over previous
import functools

import jax
import jax.numpy as jnp
from jax import lax
from jax.experimental import pallas as pl
from jax.experimental.pallas import tpu as pltpu
from jax.experimental.pallas import tpu_sc as plsc

N = 10000
E = 160000
IN_DIM = 128
HID = 64
MSG = 16
EDGE_DIM = 16
EPS = 1e-5

NC = 2
NS = 16
NW = NC * NS
CHUNK = 125
CH_W = 40
SEG = 1
CH_S = CH_W // SEG
E_S = E // SEG
E_WS = CH_S * CHUNK
NPAD = 10016
ROWS_W = NPAD // NS

BE = 1600
BN = 1000

@functools.cache
def _mesh():
    return plsc.VectorSubcoreMesh(core_axis_name="c", subcore_axis_name="s",
                                  num_cores=NC, num_subcores=NS)


def _sc_compiler_params():
    return pltpu.CompilerParams(use_tc_tiling_on_sc=False)


def _sc_gather(h, src3):
    KB = 10

    @functools.partial(
        pl.kernel,
        out_type=jax.ShapeDtypeStruct((E_S, HID), jnp.float32),
        mesh=_mesh(),
        compiler_params=_sc_compiler_params(),
        scratch_types=[
            pltpu.VMEM((CH_S, CHUNK), jnp.int32),
            pltpu.VMEM((KB, CHUNK, HID), jnp.float32),
            pltpu.SemaphoreType.DMA,
            pltpu.SemaphoreType.DMA,
        ],
    )
    def k(h_hbm, src_hbm, out_hbm, idx_v, rows_v, sem, osem):
        wid = lax.axis_index("s") * NC + lax.axis_index("c")
        pltpu.sync_copy(src_hbm.at[wid], idx_v)

        def group(g, carry):
            j0 = g * KB
            descs = [
                pltpu.async_copy(h_hbm.at[idx_v.at[j0 + b]], rows_v.at[b],
                                 sem)
                for b in range(KB)
            ]
            outs = []
            for b in range(KB):
                descs[b].wait()
                outs.append(pltpu.async_copy(
                    rows_v.at[b],
                    out_hbm.at[pl.ds((wid * CH_S + j0 + b) * CHUNK, CHUNK)],
                    osem))
            for o in outs:
                o.wait()
            return carry

        lax.fori_loop(0, CH_S // KB, group, 0)

    return k(h, src3)


def _sc_scatter(m3, dst3, zz, ones, with_count):
    KB = 10

    out_type = [jax.ShapeDtypeStruct((NC, NPAD, MSG), jnp.float32)]
    scratch = [
        pltpu.VMEM((CH_S, CHUNK), jnp.int32),
        pltpu.VMEM((E_WS, MSG), jnp.float32),
        pltpu.VMEM((CHUNK, MSG), jnp.float32),
        pltpu.VMEM_SHARED((NPAD, MSG), jnp.float32),
    ]
    if with_count:
        out_type.append(jax.ShapeDtypeStruct((NC, NPAD, MSG), jnp.float32))
        scratch.append(pltpu.VMEM_SHARED((NPAD, MSG), jnp.float32))
    scratch.append(pltpu.SemaphoreType.DMA)

    @functools.partial(
        pl.kernel,
        out_type=tuple(out_type),
        mesh=_mesh(),
        compiler_params=_sc_compiler_params(),
        scratch_types=scratch,
    )
    def k(m_hbm, dst_hbm, zz_hbm, ones_hbm, *rest):
        if with_count:
            s_out, c_out, idx_v, m_v, ones_v, shared_s, shared_c, sem = rest
        else:
            (s_out, idx_v, m_v, ones_v, shared_s, sem) = rest
        cid = lax.axis_index("c")
        sid = lax.axis_index("s")
        wid = sid * NC + cid
        r0 = sid * ROWS_W
        pltpu.sync_copy(zz_hbm.at[pl.ds(r0, ROWS_W)],
                        shared_s.at[pl.ds(r0, ROWS_W)])
        if with_count:
            pltpu.sync_copy(zz_hbm.at[pl.ds(r0, ROWS_W)],
                            shared_c.at[pl.ds(r0, ROWS_W)])
        pltpu.sync_copy(ones_hbm, ones_v)
        plsc.subcore_barrier()
        pltpu.sync_copy(dst_hbm.at[wid], idx_v)
        pltpu.sync_copy(m_hbm.at[wid], m_v)

        def body(g, carry):
            j0 = g * KB
            descs = []
            for b in range(KB):
                descs.append(pltpu.async_copy(
                    m_v.at[pl.ds((j0 + b) * CHUNK, CHUNK)],
                    shared_s.at[idx_v.at[j0 + b]], sem, add=True))
                if with_count:
                    descs.append(pltpu.async_copy(
                        ones_v, shared_c.at[idx_v.at[j0 + b]], sem,
                        add=True))
            for d in descs:
                d.wait()
            return carry

        lax.fori_loop(0, CH_S // KB, body, 0)
        plsc.subcore_barrier()
        pltpu.sync_copy(shared_s.at[pl.ds(r0, ROWS_W)],
                        s_out.at[cid, pl.ds(r0, ROWS_W)])
        if with_count:
            pltpu.sync_copy(shared_c.at[pl.ds(r0, ROWS_W)],
                            c_out.at[cid, pl.ds(r0, ROWS_W)])

    return k(m3, dst3, zz, ones)


def _tc_input_proj(x, W, b):
    def body(x_ref, w_ref, b_ref, o_ref):
        o_ref[...] = x_ref[...] @ w_ref[...] + b_ref[...]

    return pl.pallas_call(
        body,
        grid=(N // BN,),
        in_specs=[
            pl.BlockSpec((BN, IN_DIM), lambda i: (i, 0)),
            pl.BlockSpec((IN_DIM, HID), lambda i: (0, 0)),
            pl.BlockSpec((1, HID), lambda i: (0, 0)),
        ],
        out_specs=pl.BlockSpec((BN, HID), lambda i: (i, 0)),
        out_shape=jax.ShapeDtypeStruct((N, HID), jnp.float32),
    )(x, W, b.reshape(1, HID))


def _tc_messages(ea, xj, e1W, e1b, e2Wp, b2r, sel, seg):
    def body(ea_ref, xj_ref, w1_ref, b1_ref, w2_ref, b2_ref, sel_ref, o_ref):
        u = jnp.maximum(ea_ref[...] @ w1_ref[...] + b1_ref[...], 0.0)
        wp = jnp.dot(u.astype(jnp.bfloat16), w2_ref[...],
                     preferred_element_type=jnp.float32
                     ).astype(jnp.bfloat16)
        xj = xj_ref[...]
        xt = jnp.tile(xj.astype(jnp.bfloat16), (1, MSG))
        p = wp * xt
        o_ref[...] = (jnp.dot(p, sel_ref[...],
                              preferred_element_type=jnp.float32)
                      + xj @ b2_ref[...])

    nblk = E_S // BE
    return pl.pallas_call(
        body,
        grid=(nblk,),
        in_specs=[
            pl.BlockSpec((BE, EDGE_DIM), lambda i: (seg * nblk + i, 0)),
            pl.BlockSpec((BE, HID), lambda i: (i, 0)),
            pl.BlockSpec((EDGE_DIM, HID), lambda i: (0, 0)),
            pl.BlockSpec((1, HID), lambda i: (0, 0)),
            pl.BlockSpec((HID, MSG * HID), lambda i: (0, 0)),
            pl.BlockSpec((HID, MSG), lambda i: (0, 0)),
            pl.BlockSpec((MSG * HID, MSG), lambda i: (0, 0)),
        ],
        out_specs=pl.BlockSpec((BE, MSG), lambda i: (i, 0)),
        out_shape=jax.ShapeDtypeStruct((E_S, MSG), jnp.float32),
    )(ea, xj, e1W, e1b.reshape(1, HID), e2Wp.astype(jnp.bfloat16),
      b2r, sel)


def _tc_node_update(h, s_part, c_part, rW, cb, mp_W, mp_b, scale, shift,
                    final_W=None, final_b=None):
    last = final_W is not None
    out_dim = final_W.shape[1] if last else HID
    np_ = s_part.shape[0]
    nc_ = c_part.shape[0]

    def body(h_ref, s_ref, c_ref, rw_ref, cb_ref, mw_ref, mb_ref,
             sc_ref, sh_ref, *rest):
        h = h_ref[...]
        s = jnp.sum(s_ref[...], axis=0)
        cnt = jnp.maximum(jnp.sum(c_ref[...], axis=0), 1.0)
        agg = s / cnt
        xm = agg + h @ rw_ref[...] + cb_ref[...]
        h2 = h + xm @ mw_ref[...] + mb_ref[...]
        h2 = jnp.maximum(h2 * sc_ref[...] + sh_ref[...], 0.0)
        o_ref = rest[-1]
        if last:
            fw_ref, fb_ref = rest[0], rest[1]
            o_ref[...] = h2 @ fw_ref[...] + fb_ref[...]
        else:
            o_ref[...] = h2

    in_specs = [
        pl.BlockSpec((BN, HID), lambda i: (i, 0)),
        pl.BlockSpec((np_, BN, MSG), lambda i: (0, i, 0)),
        pl.BlockSpec((nc_, BN, MSG), lambda i: (0, i, 0)),
        pl.BlockSpec((HID, MSG), lambda i: (0, 0)),
        pl.BlockSpec((1, MSG), lambda i: (0, 0)),
        pl.BlockSpec((MSG, HID), lambda i: (0, 0)),
        pl.BlockSpec((1, HID), lambda i: (0, 0)),
        pl.BlockSpec((1, HID), lambda i: (0, 0)),
        pl.BlockSpec((1, HID), lambda i: (0, 0)),
    ]
    args = [h, s_part, c_part, rW, cb.reshape(1, MSG), mp_W,
            mp_b.reshape(1, HID), scale.reshape(1, HID), shift.reshape(1, HID)]
    if last:
        in_specs.append(pl.BlockSpec((HID, out_dim), lambda i: (0, 0)))
        in_specs.append(pl.BlockSpec((1, out_dim), lambda i: (0, 0)))
        args.append(final_W)
        args.append(final_b.reshape(1, out_dim))

    return pl.pallas_call(
        body,
        grid=(N // BN,),
        in_specs=in_specs,
        out_specs=pl.BlockSpec((BN, out_dim), lambda i: (i, 0)),
        out_shape=jax.ShapeDtypeStruct((N, out_dim), jnp.float32),
    )(*args)


def _permute_e2(e2W, e2b):
    Wp = e2W.reshape(HID, HID, MSG).transpose(0, 2, 1).reshape(HID, MSG * HID)
    return Wp, e2b.reshape(HID, MSG)


def kernel(x, edge_index, edge_attr, ip_W, ip_b,
           e1W_0, e1b_0, e2W_0, e2b_0, rW_0, cb_0, g_0, be_0, rm_0, rv_0,
           e1W_1, e1b_1, e2W_1, e2b_1, rW_1, cb_1, g_1, be_1, rm_1, rv_1,
           mp_W, mp_b, op_W, op_b):
    src, dst = edge_index[0], edge_index[1]
    src4 = src.reshape(SEG, NW, CH_S, CHUNK)
    dst4 = dst.reshape(SEG, NW, CH_S, CHUNK)
    ea = edge_attr
    zz = jnp.zeros((NPAD, MSG), jnp.float32)
    ones = jnp.ones((CHUNK, MSG), jnp.float32)
    sel = jnp.repeat(jnp.eye(MSG, dtype=jnp.bfloat16), HID, axis=0)

    h = _tc_input_proj(x, ip_W, ip_b)

    layers = [
        (e1W_0, e1b_0, e2W_0, e2b_0, rW_0, cb_0, g_0, be_0, rm_0, rv_0),
        (e1W_1, e1b_1, e2W_1, e2b_1, rW_1, cb_1, g_1, be_1, rm_1, rv_1),
    ]
    c_part = None
    for li, (e1W, e1b, e2W, e2b, rW, cb, g, be, rm, rv) in enumerate(layers):
        e2Wp, b2r = _permute_e2(e2W, e2b)
        scale = g / jnp.sqrt(rv + EPS)
        shift = be - rm * scale

        xj = _sc_gather(h, src4[0])
        m = _tc_messages(ea, xj, e1W, e1b, e2Wp, b2r, sel, 0)
        m3 = m.reshape(NW, E_WS, MSG)
        if li == 0:
            s_part, c_part = _sc_scatter(m3, dst4[0], zz, ones,
                                         with_count=True)
        else:
            (s_part,) = _sc_scatter(m3, dst4[0], zz, ones,
                                    with_count=False)

        if li == 0:
            h = _tc_node_update(h, s_part, c_part, rW, cb,
                                mp_W, mp_b, scale, shift)
        else:
            h = _tc_node_update(h, s_part, c_part, rW, cb,
                                mp_W, mp_b, scale, shift,
                                final_W=op_W, final_b=op_b)
    return h


# speedup vs baseline: 1.0999x; 1.0135x over previous
import functools

import jax
import jax.numpy as jnp
from jax import lax
from jax.experimental import pallas as pl
from jax.experimental.pallas import tpu as pltpu
from jax.experimental.pallas import tpu_sc as plsc

N = 10000
E = 160000
IN_DIM = 128
HID = 64
MSG = 16
EDGE_DIM = 16
EPS = 1e-5

NC = 2
NS = 16
NW = NC * NS
CHUNK = 125
CH_W = 40
SEG = 1
CH_S = CH_W // SEG
E_S = E // SEG
E_WS = CH_S * CHUNK
NPAD = 10016
ROWS_W = NPAD // NS

BE = 2000
BN = 1000

@functools.cache
def _mesh():
    return plsc.VectorSubcoreMesh(core_axis_name="c", subcore_axis_name="s",
                                  num_cores=NC, num_subcores=NS)


def _sc_compiler_params():
    return pltpu.CompilerParams(use_tc_tiling_on_sc=False)


def _sc_gather(h, src3):
    KB = 10

    @functools.partial(
        pl.kernel,
        out_type=jax.ShapeDtypeStruct((E_S, HID), jnp.float32),
        mesh=_mesh(),
        compiler_params=_sc_compiler_params(),
        scratch_types=[
            pltpu.VMEM((CH_S, CHUNK), jnp.int32),
            pltpu.VMEM((KB, CHUNK, HID), jnp.float32),
            pltpu.SemaphoreType.DMA,
            pltpu.SemaphoreType.DMA,
        ],
    )
    def k(h_hbm, src_hbm, out_hbm, idx_v, rows_v, sem, osem):
        wid = lax.axis_index("s") * NC + lax.axis_index("c")
        pltpu.sync_copy(src_hbm.at[wid], idx_v)

        def group(g, carry):
            j0 = g * KB
            descs = [
                pltpu.async_copy(h_hbm.at[idx_v.at[j0 + b]], rows_v.at[b],
                                 sem)
                for b in range(KB)
            ]
            outs = []
            for b in range(KB):
                descs[b].wait()
                outs.append(pltpu.async_copy(
                    rows_v.at[b],
                    out_hbm.at[pl.ds((wid * CH_S + j0 + b) * CHUNK, CHUNK)],
                    osem))
            for o in outs:
                o.wait()
            return carry

        lax.fori_loop(0, CH_S // KB, group, 0)

    return k(h, src3)


def _sc_scatter(m3, dst3, zz, ones, with_count):
    KB = 10

    out_type = [jax.ShapeDtypeStruct((NC, NPAD, MSG), jnp.float32)]
    scratch = [
        pltpu.VMEM((CH_S, CHUNK), jnp.int32),
        pltpu.VMEM((E_WS, MSG), jnp.float32),
        pltpu.VMEM((CHUNK, MSG), jnp.float32),
        pltpu.VMEM_SHARED((NPAD, MSG), jnp.float32),
    ]
    if with_count:
        out_type.append(jax.ShapeDtypeStruct((NC, NPAD, MSG), jnp.float32))
        scratch.append(pltpu.VMEM_SHARED((NPAD, MSG), jnp.float32))
    scratch.append(pltpu.SemaphoreType.DMA)

    @functools.partial(
        pl.kernel,
        out_type=tuple(out_type),
        mesh=_mesh(),
        compiler_params=_sc_compiler_params(),
        scratch_types=scratch,
    )
    def k(m_hbm, dst_hbm, zz_hbm, ones_hbm, *rest):
        if with_count:
            s_out, c_out, idx_v, m_v, ones_v, shared_s, shared_c, sem = rest
        else:
            (s_out, idx_v, m_v, ones_v, shared_s, sem) = rest
        cid = lax.axis_index("c")
        sid = lax.axis_index("s")
        wid = sid * NC + cid
        r0 = sid * ROWS_W
        pltpu.sync_copy(zz_hbm.at[pl.ds(r0, ROWS_W)],
                        shared_s.at[pl.ds(r0, ROWS_W)])
        if with_count:
            pltpu.sync_copy(zz_hbm.at[pl.ds(r0, ROWS_W)],
                            shared_c.at[pl.ds(r0, ROWS_W)])
        pltpu.sync_copy(ones_hbm, ones_v)
        plsc.subcore_barrier()
        pltpu.sync_copy(dst_hbm.at[wid], idx_v)
        pltpu.sync_copy(m_hbm.at[wid], m_v)

        def body(g, carry):
            j0 = g * KB
            descs = []
            for b in range(KB):
                descs.append(pltpu.async_copy(
                    m_v.at[pl.ds((j0 + b) * CHUNK, CHUNK)],
                    shared_s.at[idx_v.at[j0 + b]], sem, add=True))
                if with_count:
                    descs.append(pltpu.async_copy(
                        ones_v, shared_c.at[idx_v.at[j0 + b]], sem,
                        add=True))
            for d in descs:
                d.wait()
            return carry

        lax.fori_loop(0, CH_S // KB, body, 0)
        plsc.subcore_barrier()
        pltpu.sync_copy(shared_s.at[pl.ds(r0, ROWS_W)],
                        s_out.at[cid, pl.ds(r0, ROWS_W)])
        if with_count:
            pltpu.sync_copy(shared_c.at[pl.ds(r0, ROWS_W)],
                            c_out.at[cid, pl.ds(r0, ROWS_W)])

    return k(m3, dst3, zz, ones)


def _tc_input_proj(x, W, b):
    def body(x_ref, w_ref, b_ref, o_ref):
        o_ref[...] = x_ref[...] @ w_ref[...] + b_ref[...]

    return pl.pallas_call(
        body,
        grid=(N // BN,),
        in_specs=[
            pl.BlockSpec((BN, IN_DIM), lambda i: (i, 0)),
            pl.BlockSpec((IN_DIM, HID), lambda i: (0, 0)),
            pl.BlockSpec((1, HID), lambda i: (0, 0)),
        ],
        out_specs=pl.BlockSpec((BN, HID), lambda i: (i, 0)),
        out_shape=jax.ShapeDtypeStruct((N, HID), jnp.float32),
    )(x, W, b.reshape(1, HID))


def _tc_messages(ea, xj, e1W, e1b, e2Wp, b2r, sel, seg):
    def body(ea_ref, xj_ref, w1_ref, b1_ref, w2_ref, b2_ref, sel_ref, o_ref):
        u = jnp.maximum(ea_ref[...] @ w1_ref[...] + b1_ref[...], 0.0)
        wp = jnp.dot(u.astype(jnp.bfloat16), w2_ref[...],
                     preferred_element_type=jnp.float32
                     ).astype(jnp.bfloat16)
        xj = xj_ref[...]
        xt = jnp.tile(xj.astype(jnp.bfloat16), (1, MSG))
        p = wp * xt
        o_ref[...] = (jnp.dot(p, sel_ref[...],
                              preferred_element_type=jnp.float32)
                      + xj @ b2_ref[...])

    nblk = E_S // BE
    return pl.pallas_call(
        body,
        grid=(nblk,),
        in_specs=[
            pl.BlockSpec((BE, EDGE_DIM), lambda i: (seg * nblk + i, 0)),
            pl.BlockSpec((BE, HID), lambda i: (i, 0)),
            pl.BlockSpec((EDGE_DIM, HID), lambda i: (0, 0)),
            pl.BlockSpec((1, HID), lambda i: (0, 0)),
            pl.BlockSpec((HID, MSG * HID), lambda i: (0, 0)),
            pl.BlockSpec((HID, MSG), lambda i: (0, 0)),
            pl.BlockSpec((MSG * HID, MSG), lambda i: (0, 0)),
        ],
        out_specs=pl.BlockSpec((BE, MSG), lambda i: (i, 0)),
        out_shape=jax.ShapeDtypeStruct((E_S, MSG), jnp.float32),
    )(ea, xj, e1W, e1b.reshape(1, HID), e2Wp.astype(jnp.bfloat16),
      b2r, sel)


def _tc_node_update(h, s_part, c_part, rW, cb, mp_W, mp_b, scale, shift,
                    final_W=None, final_b=None):
    last = final_W is not None
    out_dim = final_W.shape[1] if last else HID
    np_ = s_part.shape[0]
    nc_ = c_part.shape[0]

    def body(h_ref, s_ref, c_ref, rw_ref, cb_ref, mw_ref, mb_ref,
             sc_ref, sh_ref, *rest):
        h = h_ref[...]
        s = jnp.sum(s_ref[...], axis=0)
        cnt = jnp.maximum(jnp.sum(c_ref[...], axis=0), 1.0)
        agg = s / cnt
        xm = agg + h @ rw_ref[...] + cb_ref[...]
        h2 = h + xm @ mw_ref[...] + mb_ref[...]
        h2 = jnp.maximum(h2 * sc_ref[...] + sh_ref[...], 0.0)
        o_ref = rest[-1]
        if last:
            fw_ref, fb_ref = rest[0], rest[1]
            o_ref[...] = h2 @ fw_ref[...] + fb_ref[...]
        else:
            o_ref[...] = h2

    in_specs = [
        pl.BlockSpec((BN, HID), lambda i: (i, 0)),
        pl.BlockSpec((np_, BN, MSG), lambda i: (0, i, 0)),
        pl.BlockSpec((nc_, BN, MSG), lambda i: (0, i, 0)),
        pl.BlockSpec((HID, MSG), lambda i: (0, 0)),
        pl.BlockSpec((1, MSG), lambda i: (0, 0)),
        pl.BlockSpec((MSG, HID), lambda i: (0, 0)),
        pl.BlockSpec((1, HID), lambda i: (0, 0)),
        pl.BlockSpec((1, HID), lambda i: (0, 0)),
        pl.BlockSpec((1, HID), lambda i: (0, 0)),
    ]
    args = [h, s_part, c_part, rW, cb.reshape(1, MSG), mp_W,
            mp_b.reshape(1, HID), scale.reshape(1, HID), shift.reshape(1, HID)]
    if last:
        in_specs.append(pl.BlockSpec((HID, out_dim), lambda i: (0, 0)))
        in_specs.append(pl.BlockSpec((1, out_dim), lambda i: (0, 0)))
        args.append(final_W)
        args.append(final_b.reshape(1, out_dim))

    return pl.pallas_call(
        body,
        grid=(N // BN,),
        in_specs=in_specs,
        out_specs=pl.BlockSpec((BN, out_dim), lambda i: (i, 0)),
        out_shape=jax.ShapeDtypeStruct((N, out_dim), jnp.float32),
    )(*args)


def _permute_e2(e2W, e2b):
    Wp = e2W.reshape(HID, HID, MSG).transpose(0, 2, 1).reshape(HID, MSG * HID)
    return Wp, e2b.reshape(HID, MSG)


def kernel(x, edge_index, edge_attr, ip_W, ip_b,
           e1W_0, e1b_0, e2W_0, e2b_0, rW_0, cb_0, g_0, be_0, rm_0, rv_0,
           e1W_1, e1b_1, e2W_1, e2b_1, rW_1, cb_1, g_1, be_1, rm_1, rv_1,
           mp_W, mp_b, op_W, op_b):
    src, dst = edge_index[0], edge_index[1]
    src4 = src.reshape(SEG, NW, CH_S, CHUNK)
    dst4 = dst.reshape(SEG, NW, CH_S, CHUNK)
    ea = edge_attr
    zz = jnp.zeros((NPAD, MSG), jnp.float32)
    ones = jnp.ones((CHUNK, MSG), jnp.float32)
    sel = jnp.repeat(jnp.eye(MSG, dtype=jnp.bfloat16), HID, axis=0)

    h = _tc_input_proj(x, ip_W, ip_b)

    layers = [
        (e1W_0, e1b_0, e2W_0, e2b_0, rW_0, cb_0, g_0, be_0, rm_0, rv_0),
        (e1W_1, e1b_1, e2W_1, e2b_1, rW_1, cb_1, g_1, be_1, rm_1, rv_1),
    ]
    c_part = None
    for li, (e1W, e1b, e2W, e2b, rW, cb, g, be, rm, rv) in enumerate(layers):
        e2Wp, b2r = _permute_e2(e2W, e2b)
        scale = g / jnp.sqrt(rv + EPS)
        shift = be - rm * scale

        xj = _sc_gather(h, src4[0])
        m = _tc_messages(ea, xj, e1W, e1b, e2Wp, b2r, sel, 0)
        m3 = m.reshape(NW, E_WS, MSG)
        if li == 0:
            s_part, c_part = _sc_scatter(m3, dst4[0], zz, ones,
                                         with_count=True)
        else:
            (s_part,) = _sc_scatter(m3, dst4[0], zz, ones,
                                    with_count=False)

        if li == 0:
            h = _tc_node_update(h, s_part, c_part, rW, cb,
                                mp_W, mp_b, scale, shift)
        else:
            h = _tc_node_update(h, s_part, c_part, rW, cb,
                                mp_W, mp_b, scale, shift,
                                final_W=op_W, final_b=op_b)
    return h


# speedup vs baseline: 1.1294x; 1.0268x over previous
import functools

import jax
import jax.numpy as jnp
from jax import lax
from jax.experimental import pallas as pl
from jax.experimental.pallas import tpu as pltpu
from jax.experimental.pallas import tpu_sc as plsc

N = 10000
E = 160000
IN_DIM = 128
HID = 64
MSG = 16
EDGE_DIM = 16
EPS = 1e-5

NC = 2
NS = 16
NW = NC * NS
CHUNK = 125
CH_W = 40
SEG = 1
CH_S = CH_W // SEG
E_S = E // SEG
E_WS = CH_S * CHUNK
NPAD = 10016
ROWS_W = NPAD // NS

BE = 4000
BN = 1000

@functools.cache
def _mesh():
    return plsc.VectorSubcoreMesh(core_axis_name="c", subcore_axis_name="s",
                                  num_cores=NC, num_subcores=NS)


def _sc_compiler_params():
    return pltpu.CompilerParams(use_tc_tiling_on_sc=False)


def _sc_gather(h, src3):
    KB = 10

    @functools.partial(
        pl.kernel,
        out_type=jax.ShapeDtypeStruct((E_S, HID), jnp.float32),
        mesh=_mesh(),
        compiler_params=_sc_compiler_params(),
        scratch_types=[
            pltpu.VMEM((CH_S, CHUNK), jnp.int32),
            pltpu.VMEM((KB, CHUNK, HID), jnp.float32),
            pltpu.SemaphoreType.DMA,
            pltpu.SemaphoreType.DMA,
        ],
    )
    def k(h_hbm, src_hbm, out_hbm, idx_v, rows_v, sem, osem):
        wid = lax.axis_index("s") * NC + lax.axis_index("c")
        pltpu.sync_copy(src_hbm.at[wid], idx_v)

        def group(g, carry):
            j0 = g * KB
            descs = [
                pltpu.async_copy(h_hbm.at[idx_v.at[j0 + b]], rows_v.at[b],
                                 sem)
                for b in range(KB)
            ]
            outs = []
            for b in range(KB):
                descs[b].wait()
                outs.append(pltpu.async_copy(
                    rows_v.at[b],
                    out_hbm.at[pl.ds((wid * CH_S + j0 + b) * CHUNK, CHUNK)],
                    osem))
            for o in outs:
                o.wait()
            return carry

        lax.fori_loop(0, CH_S // KB, group, 0)

    return k(h, src3)


def _sc_scatter(m3, dst3, zz, ones, with_count):
    KB = 10

    out_type = [jax.ShapeDtypeStruct((NC, NPAD, MSG), jnp.float32)]
    scratch = [
        pltpu.VMEM((CH_S, CHUNK), jnp.int32),
        pltpu.VMEM((E_WS, MSG), jnp.float32),
        pltpu.VMEM((CHUNK, MSG), jnp.float32),
        pltpu.VMEM_SHARED((NPAD, MSG), jnp.float32),
    ]
    if with_count:
        out_type.append(jax.ShapeDtypeStruct((NC, NPAD, MSG), jnp.float32))
        scratch.append(pltpu.VMEM_SHARED((NPAD, MSG), jnp.float32))
    scratch.append(pltpu.SemaphoreType.DMA)

    @functools.partial(
        pl.kernel,
        out_type=tuple(out_type),
        mesh=_mesh(),
        compiler_params=_sc_compiler_params(),
        scratch_types=scratch,
    )
    def k(m_hbm, dst_hbm, zz_hbm, ones_hbm, *rest):
        if with_count:
            s_out, c_out, idx_v, m_v, ones_v, shared_s, shared_c, sem = rest
        else:
            (s_out, idx_v, m_v, ones_v, shared_s, sem) = rest
        cid = lax.axis_index("c")
        sid = lax.axis_index("s")
        wid = sid * NC + cid
        r0 = sid * ROWS_W
        pltpu.sync_copy(zz_hbm.at[pl.ds(r0, ROWS_W)],
                        shared_s.at[pl.ds(r0, ROWS_W)])
        if with_count:
            pltpu.sync_copy(zz_hbm.at[pl.ds(r0, ROWS_W)],
                            shared_c.at[pl.ds(r0, ROWS_W)])
        pltpu.sync_copy(ones_hbm, ones_v)
        plsc.subcore_barrier()
        pltpu.sync_copy(dst_hbm.at[wid], idx_v)
        pltpu.sync_copy(m_hbm.at[wid], m_v)

        def body(g, carry):
            j0 = g * KB
            descs = []
            for b in range(KB):
                descs.append(pltpu.async_copy(
                    m_v.at[pl.ds((j0 + b) * CHUNK, CHUNK)],
                    shared_s.at[idx_v.at[j0 + b]], sem, add=True))
                if with_count:
                    descs.append(pltpu.async_copy(
                        ones_v, shared_c.at[idx_v.at[j0 + b]], sem,
                        add=True))
            for d in descs:
                d.wait()
            return carry

        lax.fori_loop(0, CH_S // KB, body, 0)
        plsc.subcore_barrier()
        pltpu.sync_copy(shared_s.at[pl.ds(r0, ROWS_W)],
                        s_out.at[cid, pl.ds(r0, ROWS_W)])
        if with_count:
            pltpu.sync_copy(shared_c.at[pl.ds(r0, ROWS_W)],
                            c_out.at[cid, pl.ds(r0, ROWS_W)])

    return k(m3, dst3, zz, ones)


def _tc_input_proj(x, W, b):
    def body(x_ref, w_ref, b_ref, o_ref):
        o_ref[...] = x_ref[...] @ w_ref[...] + b_ref[...]

    return pl.pallas_call(
        body,
        grid=(N // BN,),
        in_specs=[
            pl.BlockSpec((BN, IN_DIM), lambda i: (i, 0)),
            pl.BlockSpec((IN_DIM, HID), lambda i: (0, 0)),
            pl.BlockSpec((1, HID), lambda i: (0, 0)),
        ],
        out_specs=pl.BlockSpec((BN, HID), lambda i: (i, 0)),
        out_shape=jax.ShapeDtypeStruct((N, HID), jnp.float32),
    )(x, W, b.reshape(1, HID))


def _tc_messages(ea, xj, e1W, e1b, e2Wp, b2r, sel, seg):
    def body(ea_ref, xj_ref, w1_ref, b1_ref, w2_ref, b2_ref, sel_ref, o_ref):
        u = jnp.maximum(ea_ref[...] @ w1_ref[...] + b1_ref[...], 0.0)
        wp = jnp.dot(u.astype(jnp.bfloat16), w2_ref[...],
                     preferred_element_type=jnp.float32
                     ).astype(jnp.bfloat16)
        xj = xj_ref[...]
        xt = jnp.tile(xj.astype(jnp.bfloat16), (1, MSG))
        p = wp * xt
        o_ref[...] = (jnp.dot(p, sel_ref[...],
                              preferred_element_type=jnp.float32)
                      + xj @ b2_ref[...])

    nblk = E_S // BE
    return pl.pallas_call(
        body,
        grid=(nblk,),
        in_specs=[
            pl.BlockSpec((BE, EDGE_DIM), lambda i: (seg * nblk + i, 0)),
            pl.BlockSpec((BE, HID), lambda i: (i, 0)),
            pl.BlockSpec((EDGE_DIM, HID), lambda i: (0, 0)),
            pl.BlockSpec((1, HID), lambda i: (0, 0)),
            pl.BlockSpec((HID, MSG * HID), lambda i: (0, 0)),
            pl.BlockSpec((HID, MSG), lambda i: (0, 0)),
            pl.BlockSpec((MSG * HID, MSG), lambda i: (0, 0)),
        ],
        out_specs=pl.BlockSpec((BE, MSG), lambda i: (i, 0)),
        out_shape=jax.ShapeDtypeStruct((E_S, MSG), jnp.float32),
    )(ea, xj, e1W, e1b.reshape(1, HID), e2Wp.astype(jnp.bfloat16),
      b2r, sel)


def _tc_node_update(h, s_part, c_part, rW, cb, mp_W, mp_b, scale, shift,
                    final_W=None, final_b=None):
    last = final_W is not None
    out_dim = final_W.shape[1] if last else HID
    np_ = s_part.shape[0]
    nc_ = c_part.shape[0]

    def body(h_ref, s_ref, c_ref, rw_ref, cb_ref, mw_ref, mb_ref,
             sc_ref, sh_ref, *rest):
        h = h_ref[...]
        s = jnp.sum(s_ref[...], axis=0)
        cnt = jnp.maximum(jnp.sum(c_ref[...], axis=0), 1.0)
        agg = s / cnt
        xm = agg + h @ rw_ref[...] + cb_ref[...]
        h2 = h + xm @ mw_ref[...] + mb_ref[...]
        h2 = jnp.maximum(h2 * sc_ref[...] + sh_ref[...], 0.0)
        o_ref = rest[-1]
        if last:
            fw_ref, fb_ref = rest[0], rest[1]
            o_ref[...] = h2 @ fw_ref[...] + fb_ref[...]
        else:
            o_ref[...] = h2

    in_specs = [
        pl.BlockSpec((BN, HID), lambda i: (i, 0)),
        pl.BlockSpec((np_, BN, MSG), lambda i: (0, i, 0)),
        pl.BlockSpec((nc_, BN, MSG), lambda i: (0, i, 0)),
        pl.BlockSpec((HID, MSG), lambda i: (0, 0)),
        pl.BlockSpec((1, MSG), lambda i: (0, 0)),
        pl.BlockSpec((MSG, HID), lambda i: (0, 0)),
        pl.BlockSpec((1, HID), lambda i: (0, 0)),
        pl.BlockSpec((1, HID), lambda i: (0, 0)),
        pl.BlockSpec((1, HID), lambda i: (0, 0)),
    ]
    args = [h, s_part, c_part, rW, cb.reshape(1, MSG), mp_W,
            mp_b.reshape(1, HID), scale.reshape(1, HID), shift.reshape(1, HID)]
    if last:
        in_specs.append(pl.BlockSpec((HID, out_dim), lambda i: (0, 0)))
        in_specs.append(pl.BlockSpec((1, out_dim), lambda i: (0, 0)))
        args.append(final_W)
        args.append(final_b.reshape(1, out_dim))

    return pl.pallas_call(
        body,
        grid=(N // BN,),
        in_specs=in_specs,
        out_specs=pl.BlockSpec((BN, out_dim), lambda i: (i, 0)),
        out_shape=jax.ShapeDtypeStruct((N, out_dim), jnp.float32),
    )(*args)


def _permute_e2(e2W, e2b):
    Wp = e2W.reshape(HID, HID, MSG).transpose(0, 2, 1).reshape(HID, MSG * HID)
    return Wp, e2b.reshape(HID, MSG)


def kernel(x, edge_index, edge_attr, ip_W, ip_b,
           e1W_0, e1b_0, e2W_0, e2b_0, rW_0, cb_0, g_0, be_0, rm_0, rv_0,
           e1W_1, e1b_1, e2W_1, e2b_1, rW_1, cb_1, g_1, be_1, rm_1, rv_1,
           mp_W, mp_b, op_W, op_b):
    src, dst = edge_index[0], edge_index[1]
    src4 = src.reshape(SEG, NW, CH_S, CHUNK)
    dst4 = dst.reshape(SEG, NW, CH_S, CHUNK)
    ea = edge_attr
    zz = jnp.zeros((NPAD, MSG), jnp.float32)
    ones = jnp.ones((CHUNK, MSG), jnp.float32)
    sel = jnp.repeat(jnp.eye(MSG, dtype=jnp.bfloat16), HID, axis=0)

    h = _tc_input_proj(x, ip_W, ip_b)

    layers = [
        (e1W_0, e1b_0, e2W_0, e2b_0, rW_0, cb_0, g_0, be_0, rm_0, rv_0),
        (e1W_1, e1b_1, e2W_1, e2b_1, rW_1, cb_1, g_1, be_1, rm_1, rv_1),
    ]
    c_part = None
    for li, (e1W, e1b, e2W, e2b, rW, cb, g, be, rm, rv) in enumerate(layers):
        e2Wp, b2r = _permute_e2(e2W, e2b)
        scale = g / jnp.sqrt(rv + EPS)
        shift = be - rm * scale

        xj = _sc_gather(h, src4[0])
        m = _tc_messages(ea, xj, e1W, e1b, e2Wp, b2r, sel, 0)
        m3 = m.reshape(NW, E_WS, MSG)
        if li == 0:
            s_part, c_part = _sc_scatter(m3, dst4[0], zz, ones,
                                         with_count=True)
        else:
            (s_part,) = _sc_scatter(m3, dst4[0], zz, ones,
                                    with_count=False)

        if li == 0:
            h = _tc_node_update(h, s_part, c_part, rW, cb,
                                mp_W, mp_b, scale, shift)
        else:
            h = _tc_node_update(h, s_part, c_part, rW, cb,
                                mp_W, mp_b, scale, shift,
                                final_W=op_W, final_b=op_b)
    return h


# speedup vs baseline: 1.1302x; 1.0007x over previous
import functools

import jax
import jax.numpy as jnp
from jax import lax
from jax.experimental import pallas as pl
from jax.experimental.pallas import tpu as pltpu
from jax.experimental.pallas import tpu_sc as plsc

N = 10000
E = 160000
IN_DIM = 128
HID = 64
MSG = 16
EDGE_DIM = 16
EPS = 1e-5

NC = 2
NS = 16
NW = NC * NS
CHUNK = 125
CH_W = 40
SEG = 1
CH_S = CH_W // SEG
E_S = E // SEG
E_WS = CH_S * CHUNK
NPAD = 10016
ROWS_W = NPAD // NS

BE = 8000
BN = 1000

@functools.cache
def _mesh():
    return plsc.VectorSubcoreMesh(core_axis_name="c", subcore_axis_name="s",
                                  num_cores=NC, num_subcores=NS)


def _sc_compiler_params():
    return pltpu.CompilerParams(use_tc_tiling_on_sc=False)


def _sc_gather(h, src3):
    KB = 10

    @functools.partial(
        pl.kernel,
        out_type=jax.ShapeDtypeStruct((E_S, HID), jnp.float32),
        mesh=_mesh(),
        compiler_params=_sc_compiler_params(),
        scratch_types=[
            pltpu.VMEM((CH_S, CHUNK), jnp.int32),
            pltpu.VMEM((KB, CHUNK, HID), jnp.float32),
            pltpu.SemaphoreType.DMA,
            pltpu.SemaphoreType.DMA,
        ],
    )
    def k(h_hbm, src_hbm, out_hbm, idx_v, rows_v, sem, osem):
        wid = lax.axis_index("s") * NC + lax.axis_index("c")
        pltpu.sync_copy(src_hbm.at[wid], idx_v)

        def group(g, carry):
            j0 = g * KB
            descs = [
                pltpu.async_copy(h_hbm.at[idx_v.at[j0 + b]], rows_v.at[b],
                                 sem)
                for b in range(KB)
            ]
            outs = []
            for b in range(KB):
                descs[b].wait()
                outs.append(pltpu.async_copy(
                    rows_v.at[b],
                    out_hbm.at[pl.ds((wid * CH_S + j0 + b) * CHUNK, CHUNK)],
                    osem))
            for o in outs:
                o.wait()
            return carry

        lax.fori_loop(0, CH_S // KB, group, 0)

    return k(h, src3)


def _sc_scatter(m3, dst3, zz, ones, with_count):
    KB = 10

    out_type = [jax.ShapeDtypeStruct((NC, NPAD, MSG), jnp.float32)]
    scratch = [
        pltpu.VMEM((CH_S, CHUNK), jnp.int32),
        pltpu.VMEM((E_WS, MSG), jnp.float32),
        pltpu.VMEM((CHUNK, MSG), jnp.float32),
        pltpu.VMEM_SHARED((NPAD, MSG), jnp.float32),
    ]
    if with_count:
        out_type.append(jax.ShapeDtypeStruct((NC, NPAD, MSG), jnp.float32))
        scratch.append(pltpu.VMEM_SHARED((NPAD, MSG), jnp.float32))
    scratch.append(pltpu.SemaphoreType.DMA)

    @functools.partial(
        pl.kernel,
        out_type=tuple(out_type),
        mesh=_mesh(),
        compiler_params=_sc_compiler_params(),
        scratch_types=scratch,
    )
    def k(m_hbm, dst_hbm, zz_hbm, ones_hbm, *rest):
        if with_count:
            s_out, c_out, idx_v, m_v, ones_v, shared_s, shared_c, sem = rest
        else:
            (s_out, idx_v, m_v, ones_v, shared_s, sem) = rest
        cid = lax.axis_index("c")
        sid = lax.axis_index("s")
        wid = sid * NC + cid
        r0 = sid * ROWS_W
        pltpu.sync_copy(zz_hbm.at[pl.ds(r0, ROWS_W)],
                        shared_s.at[pl.ds(r0, ROWS_W)])
        if with_count:
            pltpu.sync_copy(zz_hbm.at[pl.ds(r0, ROWS_W)],
                            shared_c.at[pl.ds(r0, ROWS_W)])
        pltpu.sync_copy(ones_hbm, ones_v)
        plsc.subcore_barrier()
        pltpu.sync_copy(dst_hbm.at[wid], idx_v)
        pltpu.sync_copy(m_hbm.at[wid], m_v)

        def body(g, carry):
            j0 = g * KB
            descs = []
            for b in range(KB):
                descs.append(pltpu.async_copy(
                    m_v.at[pl.ds((j0 + b) * CHUNK, CHUNK)],
                    shared_s.at[idx_v.at[j0 + b]], sem, add=True))
                if with_count:
                    descs.append(pltpu.async_copy(
                        ones_v, shared_c.at[idx_v.at[j0 + b]], sem,
                        add=True))
            for d in descs:
                d.wait()
            return carry

        lax.fori_loop(0, CH_S // KB, body, 0)
        plsc.subcore_barrier()
        pltpu.sync_copy(shared_s.at[pl.ds(r0, ROWS_W)],
                        s_out.at[cid, pl.ds(r0, ROWS_W)])
        if with_count:
            pltpu.sync_copy(shared_c.at[pl.ds(r0, ROWS_W)],
                            c_out.at[cid, pl.ds(r0, ROWS_W)])

    return k(m3, dst3, zz, ones)


def _tc_input_proj(x, W, b):
    def body(x_ref, w_ref, b_ref, o_ref):
        o_ref[...] = x_ref[...] @ w_ref[...] + b_ref[...]

    return pl.pallas_call(
        body,
        grid=(N // BN,),
        in_specs=[
            pl.BlockSpec((BN, IN_DIM), lambda i: (i, 0)),
            pl.BlockSpec((IN_DIM, HID), lambda i: (0, 0)),
            pl.BlockSpec((1, HID), lambda i: (0, 0)),
        ],
        out_specs=pl.BlockSpec((BN, HID), lambda i: (i, 0)),
        out_shape=jax.ShapeDtypeStruct((N, HID), jnp.float32),
    )(x, W, b.reshape(1, HID))


def _tc_messages(ea, xj, e1W, e1b, e2Wp, b2r, sel, seg):
    def body(ea_ref, xj_ref, w1_ref, b1_ref, w2_ref, b2_ref, sel_ref, o_ref):
        u = jnp.maximum(ea_ref[...] @ w1_ref[...] + b1_ref[...], 0.0)
        wp = jnp.dot(u.astype(jnp.bfloat16), w2_ref[...],
                     preferred_element_type=jnp.float32
                     ).astype(jnp.bfloat16)
        xj = xj_ref[...]
        xt = jnp.tile(xj.astype(jnp.bfloat16), (1, MSG))
        p = wp * xt
        o_ref[...] = (jnp.dot(p, sel_ref[...],
                              preferred_element_type=jnp.float32)
                      + xj @ b2_ref[...])

    nblk = E_S // BE
    return pl.pallas_call(
        body,
        grid=(nblk,),
        in_specs=[
            pl.BlockSpec((BE, EDGE_DIM), lambda i: (seg * nblk + i, 0)),
            pl.BlockSpec((BE, HID), lambda i: (i, 0)),
            pl.BlockSpec((EDGE_DIM, HID), lambda i: (0, 0)),
            pl.BlockSpec((1, HID), lambda i: (0, 0)),
            pl.BlockSpec((HID, MSG * HID), lambda i: (0, 0)),
            pl.BlockSpec((HID, MSG), lambda i: (0, 0)),
            pl.BlockSpec((MSG * HID, MSG), lambda i: (0, 0)),
        ],
        out_specs=pl.BlockSpec((BE, MSG), lambda i: (i, 0)),
        out_shape=jax.ShapeDtypeStruct((E_S, MSG), jnp.float32),
    )(ea, xj, e1W, e1b.reshape(1, HID), e2Wp.astype(jnp.bfloat16),
      b2r, sel)


def _tc_node_update(h, s_part, c_part, rW, cb, mp_W, mp_b, scale, shift,
                    final_W=None, final_b=None):
    last = final_W is not None
    out_dim = final_W.shape[1] if last else HID
    np_ = s_part.shape[0]
    nc_ = c_part.shape[0]

    def body(h_ref, s_ref, c_ref, rw_ref, cb_ref, mw_ref, mb_ref,
             sc_ref, sh_ref, *rest):
        h = h_ref[...]
        s = jnp.sum(s_ref[...], axis=0)
        cnt = jnp.maximum(jnp.sum(c_ref[...], axis=0), 1.0)
        agg = s / cnt
        xm = agg + h @ rw_ref[...] + cb_ref[...]
        h2 = h + xm @ mw_ref[...] + mb_ref[...]
        h2 = jnp.maximum(h2 * sc_ref[...] + sh_ref[...], 0.0)
        o_ref = rest[-1]
        if last:
            fw_ref, fb_ref = rest[0], rest[1]
            o_ref[...] = h2 @ fw_ref[...] + fb_ref[...]
        else:
            o_ref[...] = h2

    in_specs = [
        pl.BlockSpec((BN, HID), lambda i: (i, 0)),
        pl.BlockSpec((np_, BN, MSG), lambda i: (0, i, 0)),
        pl.BlockSpec((nc_, BN, MSG), lambda i: (0, i, 0)),
        pl.BlockSpec((HID, MSG), lambda i: (0, 0)),
        pl.BlockSpec((1, MSG), lambda i: (0, 0)),
        pl.BlockSpec((MSG, HID), lambda i: (0, 0)),
        pl.BlockSpec((1, HID), lambda i: (0, 0)),
        pl.BlockSpec((1, HID), lambda i: (0, 0)),
        pl.BlockSpec((1, HID), lambda i: (0, 0)),
    ]
    args = [h, s_part, c_part, rW, cb.reshape(1, MSG), mp_W,
            mp_b.reshape(1, HID), scale.reshape(1, HID), shift.reshape(1, HID)]
    if last:
        in_specs.append(pl.BlockSpec((HID, out_dim), lambda i: (0, 0)))
        in_specs.append(pl.BlockSpec((1, out_dim), lambda i: (0, 0)))
        args.append(final_W)
        args.append(final_b.reshape(1, out_dim))

    return pl.pallas_call(
        body,
        grid=(N // BN,),
        in_specs=in_specs,
        out_specs=pl.BlockSpec((BN, out_dim), lambda i: (i, 0)),
        out_shape=jax.ShapeDtypeStruct((N, out_dim), jnp.float32),
    )(*args)


def _permute_e2(e2W, e2b):
    Wp = e2W.reshape(HID, HID, MSG).transpose(0, 2, 1).reshape(HID, MSG * HID)
    return Wp, e2b.reshape(HID, MSG)


def kernel(x, edge_index, edge_attr, ip_W, ip_b,
           e1W_0, e1b_0, e2W_0, e2b_0, rW_0, cb_0, g_0, be_0, rm_0, rv_0,
           e1W_1, e1b_1, e2W_1, e2b_1, rW_1, cb_1, g_1, be_1, rm_1, rv_1,
           mp_W, mp_b, op_W, op_b):
    src, dst = edge_index[0], edge_index[1]
    src4 = src.reshape(SEG, NW, CH_S, CHUNK)
    dst4 = dst.reshape(SEG, NW, CH_S, CHUNK)
    ea = edge_attr
    zz = jnp.zeros((NPAD, MSG), jnp.float32)
    ones = jnp.ones((CHUNK, MSG), jnp.float32)
    sel = jnp.repeat(jnp.eye(MSG, dtype=jnp.bfloat16), HID, axis=0)

    h = _tc_input_proj(x, ip_W, ip_b)

    layers = [
        (e1W_0, e1b_0, e2W_0, e2b_0, rW_0, cb_0, g_0, be_0, rm_0, rv_0),
        (e1W_1, e1b_1, e2W_1, e2b_1, rW_1, cb_1, g_1, be_1, rm_1, rv_1),
    ]
    c_part = None
    for li, (e1W, e1b, e2W, e2b, rW, cb, g, be, rm, rv) in enumerate(layers):
        e2Wp, b2r = _permute_e2(e2W, e2b)
        scale = g / jnp.sqrt(rv + EPS)
        shift = be - rm * scale

        xj = _sc_gather(h, src4[0])
        m = _tc_messages(ea, xj, e1W, e1b, e2Wp, b2r, sel, 0)
        m3 = m.reshape(NW, E_WS, MSG)
        if li == 0:
            s_part, c_part = _sc_scatter(m3, dst4[0], zz, ones,
                                         with_count=True)
        else:
            (s_part,) = _sc_scatter(m3, dst4[0], zz, ones,
                                    with_count=False)

        if li == 0:
            h = _tc_node_update(h, s_part, c_part, rW, cb,
                                mp_W, mp_b, scale, shift)
        else:
            h = _tc_node_update(h, s_part, c_part, rW, cb,
                                mp_W, mp_b, scale, shift,
                                final_W=op_W, final_b=op_b)
    return h
